# baseline (speedup 1.0000x reference)
import functools

import jax
import jax.numpy as jnp
from jax import lax
from jax.experimental import pallas as pl
from jax.experimental.pallas import tpu as pltpu
from jax.experimental.pallas import tpu_sc as plsc

T = 1638400
B = 16384
NK = 3
NC = 2
NS = 16
NW = NC * NS
VC = T // NW
S = B // NW
NV = S // 16
OPAD = NK * B + 8


def _body(o0, o1, o2, v0, v1, v2, out_v, out_l, out_o,
          lo_buf, len_buf, oof_buf, ext_bufs, tail_buf, sem, sem2):
    wid = lax.axis_index("s") * NC + lax.axis_index("c")
    s = wid * S
    lanes = lax.iota(jnp.int32, 16)

    offs = (o0, o1, o2)
    vals = (v0, v1, v2)

    vcopies = []
    for k in range(NK):
        src = vals[k].at[pl.ds(wid * VC, VC)]
        dst = out_v.at[pl.ds(k * T + wid * VC, VC)]
        vcopies.append(pltpu.async_copy(src, dst, sem))

    ext_idx = jnp.where(lanes == 0, s + S, jnp.where(lanes == 1, 0, B))
    off0 = []
    offB = []
    for k in range(NK):
        pltpu.async_copy(offs[k].at[ext_idx], ext_bufs.at[k], sem2).wait()
    for k in range(NK):
        ext_k = ext_bufs.at[k]
        off0.append(plsc.load_gather(ext_k, [jnp.full((16,), 1, jnp.int32)]))
        offB.append(plsc.load_gather(ext_k, [jnp.full((16,), 2, jnp.int32)]))

    d0 = offB[0] - off0[0]
    d1 = offB[1] - off0[1]
    c = (-off0[0], d0 - off0[1], d0 + d1 - off0[2])

    for k in range(NK):
        pltpu.sync_copy(offs[k].at[pl.ds(s, S)], lo_buf.at[pl.ds(0, S)])
        lo_buf[pl.ds(S, 16)] = ext_bufs[k]
        for i in range(NV):
            lo = lo_buf[pl.ds(i * 16, 16)]
            hi = plsc.load_gather(lo_buf, [jnp.full((16,), i * 16 + 1, jnp.int32) + lanes])
            len_buf[pl.ds(i * 16, 16)] = hi - lo
            oof_buf[pl.ds(i * 16, 16)] = lo + c[k]
        pltpu.sync_copy(len_buf, out_l.at[pl.ds(k * B + s, S)])
        pltpu.sync_copy(oof_buf, out_o.at[pl.ds(k * B + s, S)])

    @pl.when(wid == NW - 1)
    def _():
        tail_buf[...] = offB[NK - 1] + c[NK - 1]
        pltpu.sync_copy(tail_buf.at[pl.ds(0, 8)], out_o.at[pl.ds(NK * B, 8)])

    for cp in vcopies:
        cp.wait()


@jax.jit
def _kjt(o0, o1, o2, v0, v1, v2):
    mesh = plsc.VectorSubcoreMesh(core_axis_name="c", subcore_axis_name="s")
    out_v, out_l, out_o = pl.kernel(
        _body,
        out_type=[
            jax.ShapeDtypeStruct((NK * T,), jnp.float32),
            jax.ShapeDtypeStruct((NK * B,), jnp.int32),
            jax.ShapeDtypeStruct((OPAD,), jnp.int32),
        ],
        mesh=mesh,
        compiler_params=pltpu.CompilerParams(needs_layout_passes=False),
        scratch_types=[
            pltpu.VMEM((S + 16,), jnp.int32),
            pltpu.VMEM((S,), jnp.int32),
            pltpu.VMEM((S,), jnp.int32),
            pltpu.VMEM((NK, 16), jnp.int32),
            pltpu.VMEM((16,), jnp.int32),
            pltpu.SemaphoreType.DMA,
            pltpu.SemaphoreType.DMA,
        ],
    )(o0, o1, o2, v0, v1, v2)
    return out_v, out_l, out_o[: NK * B + 1]


def kernel(feat_0__values, feat_0__offsets, feat_1__values, feat_1__offsets,
           feat_2__values, feat_2__offsets):
    return _kjt(feat_0__offsets, feat_1__offsets, feat_2__offsets,
                feat_0__values, feat_1__values, feat_2__values)


# speedup vs baseline: 15.2731x; 15.2731x over previous
import functools

import jax
import jax.numpy as jnp
from jax import lax
from jax.experimental import pallas as pl
from jax.experimental.pallas import tpu as pltpu
from jax.experimental.pallas import tpu_sc as plsc

T = 1638400
B = 16384
NK = 3
NC = 2
NS = 16
NW = NC * NS
VC = T // NW
S = B // NW
NV = S // 16
OPAD = NK * B + 8


NBUF = 4
CH = 12800
NCHK = VC // CH
NCH = NK * NCHK


def _body(o0, o1, o2, v0, v1, v2, out_v, out_l, out_o,
          vb, lo_buf, len_buf, oof_buf, ext_bufs, tail_buf, semi, semo, sem2):
    wid = lax.axis_index("s") * NC + lax.axis_index("c")
    s = wid * S
    lanes = lax.iota(jnp.int32, 16)

    offs = (o0, o1, o2)
    vals = (v0, v1, v2)

    def start_in(idx):
        k, cc = divmod(idx, NCHK)
        src = vals[k].at[pl.ds(wid * VC + cc * CH, CH)]
        return pltpu.async_copy(src, vb.at[idx % NBUF], semi)

    def start_out(idx):
        k, cc = divmod(idx, NCHK)
        dst = out_v.at[pl.ds(k * T + wid * VC + cc * CH, CH)]
        return pltpu.async_copy(vb.at[idx % NBUF], dst, semo)

    ind = {}
    outd = {}
    for j in range(NBUF):
        ind[j] = start_in(j)

    for idx in range(NCH):
        ind[idx].wait()
        outd[idx] = start_out(idx)
        j = idx + NBUF
        if j < NCH:
            outd[j - NBUF].wait()
            ind[j] = start_in(j)

    ext_idx = jnp.where(lanes == 0, s + S, jnp.where(lanes == 1, 0, B))
    off0 = []
    offB = []
    for k in range(NK):
        pltpu.async_copy(offs[k].at[ext_idx], ext_bufs.at[k], sem2).wait()
    for k in range(NK):
        ext_k = ext_bufs.at[k]
        off0.append(plsc.load_gather(ext_k, [jnp.full((16,), 1, jnp.int32)]))
        offB.append(plsc.load_gather(ext_k, [jnp.full((16,), 2, jnp.int32)]))

    d0 = offB[0] - off0[0]
    d1 = offB[1] - off0[1]
    c = (-off0[0], d0 - off0[1], d0 + d1 - off0[2])

    for k in range(NK):
        pltpu.sync_copy(offs[k].at[pl.ds(s, S)], lo_buf.at[pl.ds(0, S)])
        lo_buf[pl.ds(S, 16)] = ext_bufs[k]
        for i in range(NV):
            lo = lo_buf[pl.ds(i * 16, 16)]
            hi = plsc.load_gather(lo_buf, [jnp.full((16,), i * 16 + 1, jnp.int32) + lanes])
            len_buf[pl.ds(i * 16, 16)] = hi - lo
            oof_buf[pl.ds(i * 16, 16)] = lo + c[k]
        pltpu.sync_copy(len_buf, out_l.at[pl.ds(k * B + s, S)])
        pltpu.sync_copy(oof_buf, out_o.at[pl.ds(k * B + s, S)])

    @pl.when(wid == NW - 1)
    def _():
        tail_buf[...] = offB[NK - 1] + c[NK - 1]
        pltpu.sync_copy(tail_buf.at[pl.ds(0, 8)], out_o.at[pl.ds(NK * B, 8)])

    for idx in range(max(0, NCH - NBUF), NCH):
        outd[idx].wait()


@jax.jit
def _kjt(o0, o1, o2, v0, v1, v2):
    mesh = plsc.VectorSubcoreMesh(core_axis_name="c", subcore_axis_name="s")
    out_v, out_l, out_o = pl.kernel(
        _body,
        out_type=[
            jax.ShapeDtypeStruct((NK * T,), jnp.float32),
            jax.ShapeDtypeStruct((NK * B,), jnp.int32),
            jax.ShapeDtypeStruct((OPAD,), jnp.int32),
        ],
        mesh=mesh,
        compiler_params=pltpu.CompilerParams(needs_layout_passes=False),
        scratch_types=[
            pltpu.VMEM((NBUF, CH), jnp.float32),
            pltpu.VMEM((S + 16,), jnp.int32),
            pltpu.VMEM((S,), jnp.int32),
            pltpu.VMEM((S,), jnp.int32),
            pltpu.VMEM((NK, 16), jnp.int32),
            pltpu.VMEM((16,), jnp.int32),
            pltpu.SemaphoreType.DMA,
            pltpu.SemaphoreType.DMA,
            pltpu.SemaphoreType.DMA,
        ],
    )(o0, o1, o2, v0, v1, v2)
    return out_v, out_l, out_o[: NK * B + 1]


def kernel(feat_0__values, feat_0__offsets, feat_1__values, feat_1__offsets,
           feat_2__values, feat_2__offsets):
    return _kjt(feat_0__offsets, feat_1__offsets, feat_2__offsets,
                feat_0__values, feat_1__values, feat_2__values)


# speedup vs baseline: 15.9532x; 1.0445x over previous
import jax
import jax.numpy as jnp
from jax import lax
from jax.experimental import pallas as pl
from jax.experimental.pallas import tpu as pltpu
from jax.experimental.pallas import tpu_sc as plsc

T = 1638400
B = 16384
NK = 3
NC = 2
NS = 16
NW = NC * NS
VC = T // NW
S = B // NW
NV = S // 16
OPAD = NK * B + 8

NBUF = 6
LEAD = NBUF - 2
CH = 12800
NCHK = VC // CH
NCH = NK * NCHK


def _body(o0, o1, o2, v0, v1, v2, out_v, out_l, out_o,
          vb, lo_bufs, len_bufs, oof_bufs, ext_bufs, tail_buf,
          semi, semo, seme, semw, sems):
    wid = lax.axis_index("s") * NC + lax.axis_index("c")
    s = wid * S
    lanes = lax.iota(jnp.int32, 16)

    offs = (o0, o1, o2)
    vals = (v0, v1, v2)

    def start_in(idx):
        k, cc = divmod(idx, NCHK)
        src = vals[k].at[pl.ds(wid * VC + cc * CH, CH)]
        return pltpu.async_copy(src, vb.at[idx % NBUF], semi)

    def start_out(idx):
        k, cc = divmod(idx, NCHK)
        dst = out_v.at[pl.ds(k * T + wid * VC + cc * CH, CH)]
        return pltpu.async_copy(vb.at[idx % NBUF], dst, semo)

    ind = {}
    outd = {}
    for j in range(LEAD):
        ind[j] = start_in(j)

    ext_idx = jnp.where(lanes == 0, s + S, jnp.where(lanes == 1, 0, B))
    extd = [pltpu.async_copy(offs[k].at[ext_idx], ext_bufs.at[k], seme)
            for k in range(NK)]
    LB = S + 16
    wind = [pltpu.async_copy(offs[k].at[pl.ds(s, S)],
                             lo_bufs.at[pl.ds(k * LB, S)], semw)
            for k in range(NK)]

    c = [None] * NK
    small = []

    def compute_key(k):
        wind[k].wait()
        kb = k * LB
        lo_bufs[pl.ds(kb + S, 16)] = ext_bufs[k]
        for i in range(NV):
            lo = lo_bufs[pl.ds(kb + i * 16, 16)]
            hi = plsc.load_gather(
                lo_bufs, [jnp.full((16,), kb + i * 16 + 1, jnp.int32) + lanes])
            len_bufs[pl.ds(k * S + i * 16, 16)] = hi - lo
            oof_bufs[pl.ds(k * S + i * 16, 16)] = lo + c[k]
        small.append(pltpu.async_copy(len_bufs.at[pl.ds(k * S, S)],
                                      out_l.at[pl.ds(k * B + s, S)], sems))
        small.append(pltpu.async_copy(oof_bufs.at[pl.ds(k * S, S)],
                                      out_o.at[pl.ds(k * B + s, S)], sems))

    for idx in range(NCH):
        ind[idx].wait()
        outd[idx] = start_out(idx)
        j = idx + LEAD
        if j < NCH:
            if j >= NBUF:
                outd[j - NBUF].wait()
            ind[j] = start_in(j)
        if idx == 0:
            for e in extd:
                e.wait()
            off0 = [plsc.load_gather(ext_bufs.at[k], [jnp.full((16,), 1, jnp.int32)])
                    for k in range(NK)]
            offB = [plsc.load_gather(ext_bufs.at[k], [jnp.full((16,), 2, jnp.int32)])
                    for k in range(NK)]
            d0 = offB[0] - off0[0]
            d1 = offB[1] - off0[1]
            c[0] = -off0[0]
            c[1] = d0 - off0[1]
            c[2] = d0 + d1 - off0[2]
        if 1 <= idx <= NK:
            compute_key(idx - 1)

    @pl.when(wid == NW - 1)
    def _():
        tail_buf[...] = offB[NK - 1] + c[NK - 1]
        pltpu.sync_copy(tail_buf.at[pl.ds(0, 8)], out_o.at[pl.ds(NK * B, 8)])

    for idx in range(max(0, NCH - NBUF), NCH):
        outd[idx].wait()
    for d in small:
        d.wait()


@jax.jit
def _kjt(o0, o1, o2, v0, v1, v2):
    mesh = plsc.VectorSubcoreMesh(core_axis_name="c", subcore_axis_name="s")
    out_v, out_l, out_o = pl.kernel(
        _body,
        out_type=[
            jax.ShapeDtypeStruct((NK * T,), jnp.float32),
            jax.ShapeDtypeStruct((NK * B,), jnp.int32),
            jax.ShapeDtypeStruct((OPAD,), jnp.int32),
        ],
        mesh=mesh,
        compiler_params=pltpu.CompilerParams(needs_layout_passes=False),
        scratch_types=[
            pltpu.VMEM((NBUF, CH), jnp.float32),
            pltpu.VMEM((NK * (S + 16),), jnp.int32),
            pltpu.VMEM((NK * S,), jnp.int32),
            pltpu.VMEM((NK * S,), jnp.int32),
            pltpu.VMEM((NK, 16), jnp.int32),
            pltpu.VMEM((16,), jnp.int32),
            pltpu.SemaphoreType.DMA,
            pltpu.SemaphoreType.DMA,
            pltpu.SemaphoreType.DMA,
            pltpu.SemaphoreType.DMA,
            pltpu.SemaphoreType.DMA,
        ],
    )(o0, o1, o2, v0, v1, v2)
    return out_v, out_l, out_o[: NK * B + 1]


def kernel(feat_0__values, feat_0__offsets, feat_1__values, feat_1__offsets,
           feat_2__values, feat_2__offsets):
    return _kjt(feat_0__offsets, feat_1__offsets, feat_2__offsets,
                feat_0__values, feat_1__values, feat_2__values)
